# baseline (speedup 1.0000x reference)
import functools

import jax
import jax.numpy as jnp
from jax import lax
from jax.experimental import pallas as pl
from jax.experimental.pallas import tpu as pltpu
from jax.experimental.pallas import tpu_sc as plsc


@functools.lru_cache(maxsize=None)
def _make_transpose(D, V, VPAD):
    info = plsc.get_sparse_core_info()
    NC, NS, L = info.num_cores, info.num_subcores, info.num_lanes
    NW = NC * NS
    n_blk = VPAD // 128
    SUB = 4
    n_sb = n_blk // SUB
    n_tail = n_blk - n_sb * SUB
    sb_per_w = (n_sb + NW - 1) // NW
    STRIDE = SUB * 128 + 5
    mesh = plsc.VectorSubcoreMesh(core_axis_name="c", subcore_axis_name="s")

    @functools.partial(
        pl.kernel,
        mesh=mesh,
        out_type=jax.ShapeDtypeStruct((VPAD * D // 128, 128), jnp.float32),
        scratch_types=[
            pltpu.VMEM((2, D, STRIDE), jnp.float32),
            pltpu.VMEM((2, SUB * D, 128), jnp.float32),
            pltpu.SemaphoreType.DMA((2,)),
            pltpu.SemaphoreType.DMA((2,)),
        ],
        compiler_params=pltpu.CompilerParams(
            use_tc_tiling_on_sc=True, needs_layout_passes=False
        ),
    )
    def k(tbl_t, out_hbm, in_v, out_v, sem_i, sem_o):
        wid = lax.axis_index("s") * NC + lax.axis_index("c")
        lanes = lax.iota(jnp.int32, L)

        def issue_in(j, b, nsub):
            pltpu.async_copy(
                tbl_t.at[:, pl.ds(j * SUB * 128, nsub * 128)],
                in_v.at[b, :, pl.ds(0, nsub * 128)],
                sem_i.at[b],
            )

        def wait_in(b, nsub):
            pltpu.make_async_copy(
                tbl_t.at[:, pl.ds(0, nsub * 128)],
                in_v.at[b, :, pl.ds(0, nsub * 128)],
                sem_i.at[b],
            ).wait()

        def issue_out(j, b, nsub):
            pltpu.async_copy(
                out_v.at[b, pl.ds(0, nsub * D)],
                out_hbm.at[pl.ds(j * SUB * D, nsub * D)],
                sem_o.at[b],
            )

        def wait_out(b, nsub):
            pltpu.make_async_copy(
                out_v.at[b, pl.ds(0, nsub * D)],
                out_hbm.at[pl.ds(0, nsub * D)],
                sem_o.at[b],
            ).wait()

        def transpose_sub(b, sub):
            def qbody(q, carry):
                vals = []
                for di in range(4):
                    for g in range(D // 16):
                        rows = g * 16 + lanes
                        cols = jnp.full((L,), sub * 128 + 4 * q + di, jnp.int32)
                        vals.append(plsc.load_gather(in_v.at[b], [rows, cols]))
                for n, v in enumerate(vals):
                    di, g = divmod(n, D // 16)
                    out_v[b, sub * D + q, pl.ds(di * 32 + g * 16, 16)] = v
                return carry

            lax.fori_loop(0, 128 // 4, qbody, 0)

        def jth(i):
            return wid + i * NW

        def _step(j, i, b):
            wait_in(b, SUB)
            jn = jth(i + 1)

            @pl.when(jn < n_sb)
            def _():
                issue_in(jn, 1 - b, SUB)

            @pl.when(i >= 2)
            def _():
                wait_out(b, SUB)

            for sub in range(SUB):
                transpose_sub(b, sub)
            issue_out(j, b, SUB)

        issue_in(jth(0), 0, SUB)

        def body(p, carry):
            for b in range(2):
                i = 2 * p + b
                j = jth(i)

                @pl.when(j < n_sb)
                def _():
                    _step(j, i, b)

            return carry

        lax.fori_loop(0, (sb_per_w + 1) // 2, body, 0)

        n_mine = (n_sb - wid + NW - 1) // NW

        @pl.when(n_mine >= 1)
        def _():
            wait_out(lax.rem(n_mine - 1, 2), SUB)

        @pl.when(n_mine >= 2)
        def _():
            wait_out(lax.rem(n_mine, 2), SUB)

        for t in range(n_tail):
            jt = n_sb * SUB + t

            @pl.when(wid == t)
            def _():
                pltpu.sync_copy(
                    tbl_t.at[:, pl.ds(jt * 128 + wid * 0, 128)],
                    in_v.at[0, :, pl.ds(0, 128)],
                )
                transpose_sub(0, 0)
                pltpu.sync_copy(
                    out_v.at[0, pl.ds(0, D)],
                    out_hbm.at[pl.ds(jt * D, D)],
                )

    return k


@functools.lru_cache(maxsize=None)
def _make_gather(B, D, VPAD, CH, NBUF):
    info = plsc.get_sparse_core_info()
    NC, NS = info.num_cores, info.num_subcores
    NW = NC * NS
    assert B % NW == 0
    b_per_w = B // NW
    assert b_per_w % (CH * NBUF) == 0
    n_grp = b_per_w // (CH * NBUF)
    mesh = plsc.VectorSubcoreMesh(core_axis_name="c", subcore_axis_name="s")

    @functools.partial(
        pl.kernel,
        mesh=mesh,
        out_type=jax.ShapeDtypeStruct((B, D), jnp.float32),
        scratch_types=[
            pltpu.VMEM((b_per_w,), jnp.int32),
            pltpu.VMEM((NBUF, CH, D), jnp.float32),
            pltpu.SemaphoreType.DMA((NBUF,)),
            pltpu.SemaphoreType.DMA((NBUF,)),
        ],
        compiler_params=pltpu.CompilerParams(use_tc_tiling_on_sc=False),
    )
    def k(idx_hbm, table_hbm, out_hbm, idx_v, rows_v, sem_g, sem_s):
        wid = lax.axis_index("s") * NC + lax.axis_index("c")
        base = wid * b_per_w
        pltpu.sync_copy(idx_hbm.at[pl.ds(base, b_per_w)], idx_v)

        def issue_gather(i, b):
            pltpu.async_copy(
                table_hbm.at[idx_v.at[pl.ds(i * CH, CH)]],
                rows_v.at[b],
                sem_g.at[b],
            )

        def wait_gather(b):
            pltpu.make_async_copy(
                table_hbm.at[pl.ds(0, CH)], rows_v.at[b], sem_g.at[b]
            ).wait()

        def issue_store(i, b):
            pltpu.async_copy(
                rows_v.at[b], out_hbm.at[pl.ds(base + i * CH, CH)], sem_s.at[b]
            )

        def wait_store(b):
            pltpu.make_async_copy(
                rows_v.at[b], out_hbm.at[pl.ds(base, CH)], sem_s.at[b]
            ).wait()

        for b in range(NBUF):
            issue_gather(b, b)

        def body(g, carry):
            for b in range(NBUF):
                i = g * NBUF + b
                wait_gather(b)
                issue_store(i, b)

                @pl.when(g < n_grp - 1)
                def _():
                    wait_store(b)
                    issue_gather(i + NBUF, b)

            return carry

        lax.fori_loop(0, n_grp, body, 0)

        for b in range(NBUF):
            wait_store(b)

    return k


def kernel(input_seq, u, item_emb, user_emb, pos_emb):
    Bt, L = input_seq.shape
    V, D = item_emb.shape
    VPAD = (V + 127) // 128 * 128
    idx = input_seq.reshape(-1).astype(jnp.int32)
    lin = _make_transpose(D, V, VPAD)(item_emb.T)
    tbl = lin.reshape(VPAD, D)
    out = _make_gather(Bt * L, D, VPAD, 512, 5)(idx, tbl)
    return out.reshape(Bt, L, D)


# speedup vs baseline: 1.0158x; 1.0158x over previous
import functools

import jax
import jax.numpy as jnp
from jax import lax
from jax.experimental import pallas as pl
from jax.experimental.pallas import tpu as pltpu
from jax.experimental.pallas import tpu_sc as plsc


@functools.lru_cache(maxsize=None)
def _make_gather(B, D, CH, NBUF):
    info = plsc.get_sparse_core_info()
    NC, NS = info.num_cores, info.num_subcores
    NW = NC * NS
    assert B % NW == 0
    b_per_w = B // NW
    assert b_per_w % (CH * NBUF) == 0
    n_grp = b_per_w // (CH * NBUF)
    mesh = plsc.VectorSubcoreMesh(core_axis_name="c", subcore_axis_name="s")

    @functools.partial(
        pl.kernel,
        mesh=mesh,
        out_type=jax.ShapeDtypeStruct((B, D), jnp.float32),
        scratch_types=[
            pltpu.VMEM((b_per_w,), jnp.int32),
            pltpu.VMEM((NBUF, CH, D), jnp.float32),
            pltpu.SemaphoreType.DMA((NBUF,)),
            pltpu.SemaphoreType.DMA((NBUF,)),
        ],
        compiler_params=pltpu.CompilerParams(use_tc_tiling_on_sc=False),
    )
    def k(idx_hbm, table_hbm, out_hbm, idx_v, rows_v, sem_g, sem_s):
        wid = lax.axis_index("s") * NC + lax.axis_index("c")
        base = wid * b_per_w
        pltpu.sync_copy(idx_hbm.at[pl.ds(base, b_per_w)], idx_v)

        def issue_gather(i, b):
            pltpu.async_copy(
                table_hbm.at[idx_v.at[pl.ds(i * CH, CH)]],
                rows_v.at[b],
                sem_g.at[b],
            )

        def wait_gather(b):
            pltpu.make_async_copy(
                table_hbm.at[pl.ds(0, CH)], rows_v.at[b], sem_g.at[b]
            ).wait()

        def issue_store(i, b):
            pltpu.async_copy(
                rows_v.at[b], out_hbm.at[pl.ds(base + i * CH, CH)], sem_s.at[b]
            )

        def wait_store(b):
            pltpu.make_async_copy(
                rows_v.at[b], out_hbm.at[pl.ds(base, CH)], sem_s.at[b]
            ).wait()

        for b in range(NBUF):
            issue_gather(b, b)

        def body(g, carry):
            for b in range(NBUF):
                i = g * NBUF + b
                wait_gather(b)
                issue_store(i, b)

                @pl.when(g < n_grp - 1)
                def _():
                    wait_store(b)
                    issue_gather(i + NBUF, b)

            return carry

        lax.fori_loop(0, n_grp, body, 0)

        for b in range(NBUF):
            wait_store(b)

    return k


def kernel(input_seq, u, item_emb, user_emb, pos_emb):
    Bt, L = input_seq.shape
    V, D = item_emb.shape
    idx = input_seq.reshape(-1).astype(jnp.int32)
    out = _make_gather(Bt * L, D, 512, 5)(idx, item_emb)
    return out.reshape(Bt, L, D)
